# Initial kernel scaffold; baseline (speedup 1.0000x reference)
import jax
import jax.numpy as jnp
from jax.experimental import pallas as pl


def kernel(input):
    raise NotImplementedError("write your pallas kernel here")


# speedup vs baseline: 1.1907x; 1.1907x over previous
import jax
import jax.numpy as jnp
from jax.experimental import pallas as pl

NORM_PERCENT = 50


def _sumsq_body(x_ref, out_ref):
    xb = x_ref[...]
    out_ref[...] = jnp.sum(xb * xb, axis=1).reshape(1, 1, -1)


def _mask_body(k, s_ref, mask_ref):
    s = s_ref[0, :]
    n = s.shape[0]
    a = jax.lax.broadcast_in_dim(s, (n, n), (0,))
    b = jax.lax.broadcast_in_dim(s, (n, n), (1,))
    jidx = jax.lax.broadcasted_iota(jnp.int32, (n, n), 0)
    cidx = jax.lax.broadcasted_iota(jnp.int32, (n, n), 1)
    beats = (a > b) | ((a == b) & (jidx < cidx))
    rank = jnp.sum(beats.astype(jnp.int32), axis=0)
    mask_ref[0, :] = (rank < k).astype(jnp.float32)


def _mul_body(x_ref, m_ref, out_ref):
    out_ref[...] = x_ref[...] * m_ref[0, 0, :][:, None]


def kernel(input):
    x = input
    _, C, H, W = x.shape
    k = int(float(NORM_PERCENT) / 100.0 * float(C))
    if k <= 0 or k >= C:
        k = C
    HW = H * W
    CB = 8
    nblk = C // CB

    x2 = x.reshape(C, HW)

    sumsq = pl.pallas_call(
        _sumsq_body,
        grid=(nblk,),
        in_specs=[pl.BlockSpec((CB, HW), lambda i: (i, 0))],
        out_specs=pl.BlockSpec((1, 1, CB), lambda i: (i, 0, 0)),
        out_shape=jax.ShapeDtypeStruct((nblk, 1, CB), jnp.float32),
    )(x2)

    mask = pl.pallas_call(
        lambda s_ref, mask_ref: _mask_body(k, s_ref, mask_ref),
        in_specs=[pl.BlockSpec((1, C), lambda: (0, 0))],
        out_specs=pl.BlockSpec((1, C), lambda: (0, 0)),
        out_shape=jax.ShapeDtypeStruct((1, C), jnp.float32),
    )(sumsq.reshape(1, C))

    out = pl.pallas_call(
        _mul_body,
        grid=(nblk,),
        in_specs=[
            pl.BlockSpec((CB, HW), lambda i: (i, 0)),
            pl.BlockSpec((1, 1, CB), lambda i: (i, 0, 0)),
        ],
        out_specs=pl.BlockSpec((CB, HW), lambda i: (i, 0)),
        out_shape=jax.ShapeDtypeStruct((C, HW), jnp.float32),
    )(x2, mask.reshape(nblk, 1, CB))

    return out.reshape(x.shape)
